# baseline (speedup 1.0000x reference)
import functools

import jax
import jax.numpy as jnp
from jax.experimental import pallas as pl
from jax.experimental.pallas import tpu as pltpu

_CHUNK = 2048


def _enqueue_dma(sc_ref, keys_sh, queue_hbm, out_hbm, sem, fsem,
                 scr_q, scr_k, *, n, kq, blk, d):
    p = sc_ref[0]
    s8 = sc_ref[1]
    nchunks = kq // blk
    info = []
    for c in range(nchunks):
        q_mod = jax.lax.rem(c * blk - p + kq, kq)
        q = jnp.where(q_mod < n, q_mod, q_mod - kq)
        full = (q >= 0) & (q <= n - blk)
        empty = (q <= -blk) | (q >= n)
        dst = out_hbm.at[pl.ds(c * blk, blk)]
        src_off = (jnp.maximum(q + s8, 0) // 8) * 8
        cp_keys = pltpu.make_async_copy(
            keys_sh.at[pl.ds(src_off, blk)], dst, sem)
        cp_queue = pltpu.make_async_copy(
            queue_hbm.at[pl.ds(c * blk, blk)], dst, sem)
        pl.when(full)(cp_keys.start)
        pl.when(jnp.logical_not(full))(cp_queue.start)
        info.append((q, full, empty, cp_queue))
    for _, _, _, cp_queue in info:
        cp_queue.wait()

    ntiles = blk // 8
    for c in range(nchunks):
        q, full, empty, _ = info[c]
        boundary = jnp.logical_not(full | empty)

        @pl.when(boundary)
        def _fix(c=c, q=q):
            def tile_start(t, cnt):
                tq = q + 8 * t
                full_t = (tq >= 0) & (tq <= n - 8)

                @pl.when(full_t)
                def _():
                    off = (jnp.maximum(tq + s8, 0) // 8) * 8
                    pltpu.make_async_copy(
                        keys_sh.at[pl.ds(off, 8)],
                        out_hbm.at[pl.ds(c * blk + 8 * t, 8)],
                        fsem).start()
                return cnt + jnp.where(full_t, 1, 0)

            cnt = jax.lax.fori_loop(0, ntiles, tile_start, jnp.int32(0))

            def tile_wait(_, x):
                pltpu.make_async_copy(
                    keys_sh.at[pl.ds(0, 8)], out_hbm.at[pl.ds(0, 8)],
                    fsem).wait()
                return x

            jax.lax.fori_loop(0, cnt, tile_wait, jnp.int32(0))

            def partial_fix(t):
                tq = q + 8 * t
                row0 = c * blk + 8 * t
                cp1 = pltpu.make_async_copy(
                    queue_hbm.at[pl.ds(row0, 8)], scr_q, fsem)
                cp1.start()
                cp1.wait()
                off = (jnp.maximum(tq + s8, 0) // 8) * 8
                cp2 = pltpu.make_async_copy(
                    keys_sh.at[pl.ds(off, 8)], scr_k, fsem)
                cp2.start()
                cp2.wait()
                u = jax.lax.broadcasted_iota(jnp.int32, (8, 1), 0)
                valid = (tq + u >= 0) & (tq + u < n)
                scr_q[...] = jnp.where(valid, scr_k[...], scr_q[...])
                cp3 = pltpu.make_async_copy(
                    scr_q, out_hbm.at[pl.ds(row0, 8)], fsem)
                cp3.start()
                cp3.wait()

            s = -q
            e = n - q
            head = (q < 0) & (jax.lax.rem(s, 8) != 0)
            tail = (q > n - blk) & (q < n) & (jax.lax.rem(e, 8) != 0)
            pl.when(head)(lambda: partial_fix(s // 8))
            pl.when(tail)(lambda: partial_fix(e // 8))


def kernel(keys, queue, ptr):
    n, d = keys.shape
    kq = queue.shape[0]
    blk = _CHUNK
    p = jnp.asarray(ptr, jnp.int32) % kq
    s8 = p % 8
    keys_sh = jax.lax.dynamic_update_slice(
        jnp.zeros((n + 8, d), keys.dtype), keys, (s8, jnp.int32(0))
    )
    grid_spec = pltpu.PrefetchScalarGridSpec(
        num_scalar_prefetch=1,
        grid=(1,),
        in_specs=[
            pl.BlockSpec(memory_space=pltpu.MemorySpace.HBM),
            pl.BlockSpec(memory_space=pltpu.MemorySpace.HBM),
        ],
        out_specs=pl.BlockSpec(memory_space=pltpu.MemorySpace.HBM),
        scratch_shapes=[
            pltpu.SemaphoreType.DMA,
            pltpu.SemaphoreType.DMA,
            pltpu.VMEM((8, d), keys.dtype),
            pltpu.VMEM((8, d), keys.dtype),
        ],
    )
    return pl.pallas_call(
        functools.partial(_enqueue_dma, n=n, kq=kq, blk=blk, d=d),
        grid_spec=grid_spec,
        out_shape=jax.ShapeDtypeStruct((kq, d), queue.dtype),
    )(jnp.stack([p, s8]), keys_sh, queue)


# speedup vs baseline: 31.6237x; 31.6237x over previous
import functools

import jax
import jax.numpy as jnp
from jax.experimental import pallas as pl
from jax.experimental.pallas import tpu as pltpu

_BLK = 4096


def _enqueue_block(sc_ref, keys_pad_ref, queue_ref, out_ref, *, n, kq, blk):
    i = pl.program_id(0)
    r0 = i * blk
    p = sc_ref[0]
    base = sc_ref[1]
    q_mod = jax.lax.rem(r0 - p + kq, kq)
    q = jnp.where(q_mod < n, q_mod, q_mod - kq)
    overlap = (q > -blk) & (q < n)

    @pl.when(overlap)
    def _merge():
        offset = (jnp.maximum(base + q, 0) // 8) * 8
        aligned = keys_pad_ref[pl.ds(offset, blk), :]
        u = jax.lax.broadcasted_iota(jnp.int32, (blk, 1), 0)
        ki = q + u
        mask = (ki >= 0) & (ki < n)
        out_ref[...] = jnp.where(mask, aligned, queue_ref[...])

    @pl.when(jnp.logical_not(overlap))
    def _copy():
        out_ref[...] = queue_ref[...]


def kernel(keys, queue, ptr):
    n, d = keys.shape
    kq = queue.shape[0]
    blk = _BLK
    p = jnp.asarray(ptr, jnp.int32) % kq
    base = blk + p % 8
    pad_rows = n + 2 * blk + 8
    keys_pad = jax.lax.dynamic_update_slice(
        jnp.zeros((pad_rows, d), keys.dtype), keys, (base, jnp.int32(0))
    )
    grid_spec = pltpu.PrefetchScalarGridSpec(
        num_scalar_prefetch=1,
        grid=(kq // blk,),
        in_specs=[
            pl.BlockSpec((pad_rows, d), lambda i, pref: (0, 0)),
            pl.BlockSpec((blk, d), lambda i, pref: (i, 0)),
        ],
        out_specs=pl.BlockSpec((blk, d), lambda i, pref: (i, 0)),
    )
    return pl.pallas_call(
        functools.partial(_enqueue_block, n=n, kq=kq, blk=blk),
        grid_spec=grid_spec,
        out_shape=jax.ShapeDtypeStruct((kq, d), queue.dtype),
    )(jnp.stack([p, base]), keys_pad, queue)
